# baseline (speedup 1.0000x reference)
import functools

import jax
import jax.numpy as jnp
from jax import lax
from jax.experimental import pallas as pl
from jax.experimental.pallas import tpu as pltpu
from jax.experimental.pallas import tpu_sc as plsc

N = 100000
E = 1600000
F = 128
H = 32
G = 64
L = 4
C = 2

NHALF = N // 2
SROWS = NHALF + 48
CH = 128
TILES = 16
NROW = 12544
EPAD = NROW * CH - E
RPT = NROW // TILES
IDXB = 56

NOB = RPT // IDXB


def _sc_agg_body(h_hbm, src_hbm, dst_hbm, zeros_hbm, out_hbm,
                 srcb, dstd, rows, aggsh, gsem, ssem):
    c = lax.axis_index("c")
    s = lax.axis_index("s")
    off = c * NHALF

    z0 = s * (SROWS // TILES)
    pltpu.sync_copy(zeros_hbm.at[pl.ds(z0, SROWS // TILES)],
                    aggsh.at[pl.ds(z0, SROWS // TILES)])
    plsc.subcore_barrier()

    def localize(j):
        for q in range(CH // 16):
            v = dstd[j, pl.ds(q * 16, 16)]
            lv = v - off
            ok = (lv >= 0) & (lv < NHALF)
            dstd[j, pl.ds(q * 16, 16)] = jnp.where(ok, lv, NHALF)

    @pl.loop(0, NOB)
    def _ob(ob):
        r0 = s * RPT + ob * IDXB
        pltpu.sync_copy(src_hbm.at[pl.ds(r0, IDXB)], srcb)
        pltpu.sync_copy(dst_hbm.at[pl.ds(r0, IDXB)], dstd)

        for p in range(3):
            localize(p)
            pltpu.async_copy(h_hbm.at[srcb.at[p]], rows.at[p], gsem.at[p])

        @pl.loop(0, IDXB)
        def _go(j):
            b = j & 3
            pltpu.make_async_copy(h_hbm.at[srcb.at[j]], rows.at[b],
                                  gsem.at[b]).wait()
            pltpu.async_copy(rows.at[b], aggsh.at[dstd.at[j]],
                             ssem.at[b], add=True)

            @pl.when(j < IDXB - 3)
            def _():
                bn = (j + 3) & 3

                @pl.when(j >= 1)
                def _():
                    pltpu.make_async_copy(rows.at[bn],
                                          aggsh.at[dstd.at[j - 1]],
                                          ssem.at[bn]).wait()

                localize(j + 3)
                pltpu.async_copy(h_hbm.at[srcb.at[j + 3]], rows.at[bn],
                                 gsem.at[bn])

        for q in range(4):
            jq = IDXB - 4 + q
            pltpu.make_async_copy(rows.at[jq & 3],
                                  aggsh.at[dstd.at[jq]],
                                  ssem.at[jq & 3]).wait()

    plsc.subcore_barrier()
    w0 = s * (SROWS // TILES)
    pltpu.sync_copy(aggsh.at[pl.ds(w0, SROWS // TILES)],
                    out_hbm.at[c].at[pl.ds(w0, SROWS // TILES)])


_sc_agg_cached = None


def _sc_agg(h, src2, dst2, zeros):
    global _sc_agg_cached
    if _sc_agg_cached is None:
        mesh = plsc.VectorSubcoreMesh(core_axis_name="c",
                                      subcore_axis_name="s",
                                      num_cores=2, num_subcores=TILES)
        _sc_agg_cached = functools.partial(
            pl.kernel,
            out_type=jax.ShapeDtypeStruct((2, SROWS, H), jnp.float32),
            mesh=mesh,
            compiler_params=pltpu.CompilerParams(use_tc_tiling_on_sc=False),
            scratch_types=[
                pltpu.VMEM((IDXB, CH), jnp.int32),
                pltpu.VMEM((IDXB, CH), jnp.int32),
                pltpu.VMEM((4, CH, H), jnp.float32),
                pltpu.VMEM_SHARED((SROWS, H), jnp.float32),
                pltpu.SemaphoreType.DMA((4,)),
                pltpu.SemaphoreType.DMA((4,)),
            ],
        )(_sc_agg_body)
    return _sc_agg_cached(h, src2, dst2, zeros)


BN = 1000
NBLK = N // BN


def _enc_body(xr, wr, br, or_):
    or_[...] = jnp.dot(xr[...], wr[...],
                       preferred_element_type=jnp.float32) + br[...]


def _encode(x, enc_W, enc_b):
    return pl.pallas_call(
        _enc_body,
        grid=(NBLK,),
        in_specs=[pl.BlockSpec((BN, F), lambda i: (i, 0)),
                  pl.BlockSpec((F, H), lambda i: (0, 0)),
                  pl.BlockSpec((1, H), lambda i: (0, 0))],
        out_specs=pl.BlockSpec((BN, H), lambda i: (i, 0)),
        out_shape=jax.ShapeDtypeStruct((N, H), jnp.float32),
    )(x, enc_W, enc_b.reshape(1, H))


def _mlp_body(eps_ref, hr, ar, w1r, b1r, w2r, b2r, h2r, psr, pqr):
    i = pl.program_id(0)
    z = hr[...] * (1.0 + eps_ref[0]) + ar[...]
    u = jnp.maximum(
        jnp.dot(z, w1r[...], preferred_element_type=jnp.float32) + b1r[...],
        0.0)
    h2 = jnp.dot(u, w2r[...], preferred_element_type=jnp.float32) + b2r[...]
    h2r[...] = h2

    @pl.when(i == 0)
    def _():
        psr[...] = jnp.zeros_like(psr)
        pqr[...] = jnp.zeros_like(pqr)

    psr[...] += jnp.sum(h2, axis=0, keepdims=True)
    pqr[...] += jnp.sum(h2 * h2, axis=0, keepdims=True)


def _mlp(h, agg, eps_l, W1_l, b1_l, W2_l, b2_l):
    return pl.pallas_call(
        _mlp_body,
        grid=(NBLK,),
        in_specs=[pl.BlockSpec(memory_space=pltpu.SMEM),
                  pl.BlockSpec((BN, H), lambda i: (i, 0)),
                  pl.BlockSpec((BN, H), lambda i: (i, 0)),
                  pl.BlockSpec((H, H), lambda i: (0, 0)),
                  pl.BlockSpec((1, H), lambda i: (0, 0)),
                  pl.BlockSpec((H, H), lambda i: (0, 0)),
                  pl.BlockSpec((1, H), lambda i: (0, 0))],
        out_specs=[pl.BlockSpec((BN, H), lambda i: (i, 0)),
                   pl.BlockSpec((1, H), lambda i: (0, 0)),
                   pl.BlockSpec((1, H), lambda i: (0, 0))],
        out_shape=[jax.ShapeDtypeStruct((N, H), jnp.float32),
                   jax.ShapeDtypeStruct((1, H), jnp.float32),
                   jax.ShapeDtypeStruct((1, H), jnp.float32)],
    )(eps_l.reshape(1), h, agg, W1_l, b1_l.reshape(1, H), W2_l,
      b2_l.reshape(1, H))


def _norm_body(h2r, psr, pqr, gr, br, or_):
    mean = psr[...] * (1.0 / N)
    var = pqr[...] * (1.0 / N) - mean * mean
    scale = gr[...] * lax.rsqrt(var + 1e-5)
    or_[...] = jnp.maximum((h2r[...] - mean) * scale + br[...], 0.0)


def _norm(h2, ps, pq, gamma_l, beta_l):
    return pl.pallas_call(
        _norm_body,
        grid=(NBLK,),
        in_specs=[pl.BlockSpec((BN, H), lambda i: (i, 0)),
                  pl.BlockSpec((1, H), lambda i: (0, 0)),
                  pl.BlockSpec((1, H), lambda i: (0, 0)),
                  pl.BlockSpec((1, H), lambda i: (0, 0)),
                  pl.BlockSpec((1, H), lambda i: (0, 0))],
        out_specs=pl.BlockSpec((BN, H), lambda i: (i, 0)),
        out_shape=jax.ShapeDtypeStruct((N, H), jnp.float32),
    )(h2, ps, pq, gamma_l.reshape(1, H), beta_l.reshape(1, H))


def _pool_body(hr, br, wr, cbr, or_, acc):
    i = pl.program_id(0)

    @pl.when(i == 0)
    def _():
        acc[...] = jnp.zeros_like(acc)

    oh = (lax.broadcasted_iota(jnp.int32, (G, BN), 0) == br[0]).astype(
        jnp.float32)
    acc[...] += jnp.dot(oh, hr[...], preferred_element_type=jnp.float32)

    @pl.when(i == NBLK - 1)
    def _():
        or_[...] = jnp.dot(acc[...], wr[...],
                           preferred_element_type=jnp.float32) + cbr[...]


def _pool_cls(h, batch3, cls_W, cls_b):
    return pl.pallas_call(
        _pool_body,
        grid=(NBLK,),
        in_specs=[pl.BlockSpec((BN, H), lambda i: (i, 0)),
                  pl.BlockSpec((1, 1, BN), lambda i: (i, 0, 0)),
                  pl.BlockSpec((H, C), lambda i: (0, 0)),
                  pl.BlockSpec((1, C), lambda i: (0, 0))],
        out_specs=pl.BlockSpec((G, C), lambda i: (0, 0)),
        out_shape=jax.ShapeDtypeStruct((G, C), jnp.float32),
        scratch_shapes=[pltpu.VMEM((G, H), jnp.float32)],
    )(h, batch3, cls_W, cls_b.reshape(1, C))


def kernel(x, edge_index, batch, enc_W, enc_b, eps, W1, b1, W2, b2,
           gamma, beta, cls_W, cls_b):
    src2 = jnp.concatenate(
        [edge_index[0], jnp.zeros((EPAD,), jnp.int32)]).reshape(NROW, CH)
    dst2 = jnp.concatenate(
        [edge_index[1], jnp.full((EPAD,), N, jnp.int32)]).reshape(NROW, CH)
    zeros = jnp.zeros((SROWS, H), jnp.float32)
    batch3 = batch.astype(jnp.int32).reshape(NBLK, 1, BN)

    h = _encode(x, enc_W, enc_b)
    for l in range(L):
        o = _sc_agg(h, src2, dst2, zeros)
        agg = jnp.concatenate([o[0, :NHALF], o[1, :NHALF]], axis=0)
        h2, ps, pq = _mlp(h, agg, eps[l], W1[l], b1[l], W2[l], b2[l])
        h = _norm(h2, ps, pq, gamma[l], beta[l])
    return _pool_cls(h, batch3, cls_W, cls_b)


# speedup vs baseline: 2.0100x; 2.0100x over previous
import functools

import jax
import jax.numpy as jnp
from jax import lax
from jax.experimental import pallas as pl
from jax.experimental.pallas import tpu as pltpu
from jax.experimental.pallas import tpu_sc as plsc

N = 100000
E = 1600000
F = 128
H = 32
G = 64
L = 4
C = 2
HH = H // 2

SROWS = 100096
CH = 128
TILES = 16
NROW = 12544
EPAD = NROW * CH - E
RPT = NROW // TILES
IDXB = 56
NOB = RPT // IDXB
ZPT = SROWS // TILES
WPT = 6256


def _sc_agg_body(hc_hbm, src_hbm, dst_hbm, zeros_hbm, out_hbm,
                 srcb, dstd, rows, aggsh, gsem, ssem):
    c = lax.axis_index("c")
    s = lax.axis_index("s")

    z0 = s * ZPT
    pltpu.sync_copy(zeros_hbm.at[pl.ds(z0, ZPT)], aggsh.at[pl.ds(z0, ZPT)])
    plsc.subcore_barrier()

    table = hc_hbm.at[c]

    @pl.loop(0, NOB)
    def _ob(ob):
        r0 = s * RPT + ob * IDXB
        pltpu.sync_copy(src_hbm.at[pl.ds(r0, IDXB)], srcb)
        pltpu.sync_copy(dst_hbm.at[pl.ds(r0, IDXB)], dstd)

        for p in range(3):
            pltpu.async_copy(table.at[srcb.at[p]], rows.at[p], gsem.at[p])

        @pl.loop(0, IDXB)
        def _go(j):
            b = j & 3
            pltpu.make_async_copy(table.at[srcb.at[j]], rows.at[b],
                                  gsem.at[b]).wait()
            pltpu.async_copy(rows.at[b], aggsh.at[dstd.at[j]],
                             ssem.at[b], add=True)

            @pl.when(j < IDXB - 3)
            def _():
                bn = (j + 3) & 3

                @pl.when(j >= 1)
                def _():
                    pltpu.make_async_copy(rows.at[bn],
                                          aggsh.at[dstd.at[j - 1]],
                                          ssem.at[bn]).wait()

                pltpu.async_copy(table.at[srcb.at[j + 3]], rows.at[bn],
                                 gsem.at[bn])

        for q in range(4):
            jq = IDXB - 4 + q
            pltpu.make_async_copy(rows.at[jq & 3],
                                  aggsh.at[dstd.at[jq]],
                                  ssem.at[jq & 3]).wait()

    plsc.subcore_barrier()
    w0 = s * WPT

    @pl.when(s < TILES - 1)
    def _():
        pltpu.sync_copy(aggsh.at[pl.ds(w0, WPT)],
                        out_hbm.at[c].at[pl.ds(w0, WPT)])

    @pl.when(s == TILES - 1)
    def _():
        pltpu.sync_copy(aggsh.at[pl.ds(w0, N - (TILES - 1) * WPT)],
                        out_hbm.at[c].at[pl.ds(w0, N - (TILES - 1) * WPT)])


_sc_agg_cached = None


def _sc_agg(hc, src2, dst2, zeros):
    global _sc_agg_cached
    if _sc_agg_cached is None:
        mesh = plsc.VectorSubcoreMesh(core_axis_name="c",
                                      subcore_axis_name="s",
                                      num_cores=2, num_subcores=TILES)
        _sc_agg_cached = functools.partial(
            pl.kernel,
            out_type=jax.ShapeDtypeStruct((2, N, HH), jnp.float32),
            mesh=mesh,
            compiler_params=pltpu.CompilerParams(use_tc_tiling_on_sc=False),
            scratch_types=[
                pltpu.VMEM((IDXB, CH), jnp.int32),
                pltpu.VMEM((IDXB, CH), jnp.int32),
                pltpu.VMEM((4, CH, HH), jnp.float32),
                pltpu.VMEM_SHARED((SROWS, HH), jnp.float32),
                pltpu.SemaphoreType.DMA((4,)),
                pltpu.SemaphoreType.DMA((4,)),
            ],
        )(_sc_agg_body)
    return _sc_agg_cached(hc, src2, dst2, zeros)


BN = 1000
NBLK = N // BN


def _split(h):
    return jnp.stack([h[:, :HH], h[:, HH:]])


def _enc_body(xr, wr, br, or_):
    h = jnp.dot(xr[...], wr[...], preferred_element_type=jnp.float32) + br[...]
    or_[...] = _split(h)


def _encode(x, enc_W, enc_b):
    return pl.pallas_call(
        _enc_body,
        grid=(NBLK,),
        in_specs=[pl.BlockSpec((BN, F), lambda i: (i, 0)),
                  pl.BlockSpec((F, H), lambda i: (0, 0)),
                  pl.BlockSpec((1, H), lambda i: (0, 0))],
        out_specs=pl.BlockSpec((2, BN, HH), lambda i: (0, i, 0)),
        out_shape=jax.ShapeDtypeStruct((2, N, HH), jnp.float32),
    )(x, enc_W, enc_b.reshape(1, H))


def _mlp_body(eps_ref, hr, ar, w1r, b1r, w2r, b2r, h2r, psr, pqr):
    i = pl.program_id(0)
    hcat = jnp.concatenate([hr[0], hr[1]], axis=1)
    acat = jnp.concatenate([ar[0], ar[1]], axis=1)
    z = hcat * (1.0 + eps_ref[0]) + acat
    u = jnp.maximum(
        jnp.dot(z, w1r[...], preferred_element_type=jnp.float32) + b1r[...],
        0.0)
    h2 = jnp.dot(u, w2r[...], preferred_element_type=jnp.float32) + b2r[...]
    h2r[...] = h2

    @pl.when(i == 0)
    def _():
        psr[...] = jnp.zeros_like(psr)
        pqr[...] = jnp.zeros_like(pqr)

    psr[...] += jnp.sum(h2, axis=0, keepdims=True)
    pqr[...] += jnp.sum(h2 * h2, axis=0, keepdims=True)


def _mlp(hc, agg2, eps_l, W1_l, b1_l, W2_l, b2_l):
    return pl.pallas_call(
        _mlp_body,
        grid=(NBLK,),
        in_specs=[pl.BlockSpec(memory_space=pltpu.SMEM),
                  pl.BlockSpec((2, BN, HH), lambda i: (0, i, 0)),
                  pl.BlockSpec((2, BN, HH), lambda i: (0, i, 0)),
                  pl.BlockSpec((H, H), lambda i: (0, 0)),
                  pl.BlockSpec((1, H), lambda i: (0, 0)),
                  pl.BlockSpec((H, H), lambda i: (0, 0)),
                  pl.BlockSpec((1, H), lambda i: (0, 0))],
        out_specs=[pl.BlockSpec((BN, H), lambda i: (i, 0)),
                   pl.BlockSpec((1, H), lambda i: (0, 0)),
                   pl.BlockSpec((1, H), lambda i: (0, 0))],
        out_shape=[jax.ShapeDtypeStruct((N, H), jnp.float32),
                   jax.ShapeDtypeStruct((1, H), jnp.float32),
                   jax.ShapeDtypeStruct((1, H), jnp.float32)],
    )(eps_l.reshape(1), hc, agg2, W1_l, b1_l.reshape(1, H), W2_l,
      b2_l.reshape(1, H))


def _norm_body(h2r, psr, pqr, gr, br, or_):
    mean = psr[...] * (1.0 / N)
    var = pqr[...] * (1.0 / N) - mean * mean
    scale = gr[...] * lax.rsqrt(var + 1e-5)
    h = jnp.maximum((h2r[...] - mean) * scale + br[...], 0.0)
    or_[...] = _split(h)


def _norm(h2, ps, pq, gamma_l, beta_l):
    return pl.pallas_call(
        _norm_body,
        grid=(NBLK,),
        in_specs=[pl.BlockSpec((BN, H), lambda i: (i, 0)),
                  pl.BlockSpec((1, H), lambda i: (0, 0)),
                  pl.BlockSpec((1, H), lambda i: (0, 0)),
                  pl.BlockSpec((1, H), lambda i: (0, 0)),
                  pl.BlockSpec((1, H), lambda i: (0, 0))],
        out_specs=pl.BlockSpec((2, BN, HH), lambda i: (0, i, 0)),
        out_shape=jax.ShapeDtypeStruct((2, N, HH), jnp.float32),
    )(h2, ps, pq, gamma_l.reshape(1, H), beta_l.reshape(1, H))


def _pool_body(hr, br, wr, cbr, or_, acc):
    i = pl.program_id(0)

    @pl.when(i == 0)
    def _():
        acc[...] = jnp.zeros_like(acc)

    h = jnp.concatenate([hr[0], hr[1]], axis=1)
    oh = (lax.broadcasted_iota(jnp.int32, (G, BN), 0) == br[0]).astype(
        jnp.float32)
    acc[...] += jnp.dot(oh, h, preferred_element_type=jnp.float32)

    @pl.when(i == NBLK - 1)
    def _():
        or_[...] = jnp.dot(acc[...], wr[...],
                           preferred_element_type=jnp.float32) + cbr[...]


def _pool_cls(hc, batch3, cls_W, cls_b):
    return pl.pallas_call(
        _pool_body,
        grid=(NBLK,),
        in_specs=[pl.BlockSpec((2, BN, HH), lambda i: (0, i, 0)),
                  pl.BlockSpec((1, 1, BN), lambda i: (i, 0, 0)),
                  pl.BlockSpec((H, C), lambda i: (0, 0)),
                  pl.BlockSpec((1, C), lambda i: (0, 0))],
        out_specs=pl.BlockSpec((G, C), lambda i: (0, 0)),
        out_shape=jax.ShapeDtypeStruct((G, C), jnp.float32),
        scratch_shapes=[pltpu.VMEM((G, H), jnp.float32)],
    )(hc, batch3, cls_W, cls_b.reshape(1, C))


def kernel(x, edge_index, batch, enc_W, enc_b, eps, W1, b1, W2, b2,
           gamma, beta, cls_W, cls_b):
    src2 = jnp.concatenate(
        [edge_index[0], jnp.zeros((EPAD,), jnp.int32)]).reshape(NROW, CH)
    dst2 = jnp.concatenate(
        [edge_index[1], jnp.full((EPAD,), N, jnp.int32)]).reshape(NROW, CH)
    zeros = jnp.zeros((SROWS, HH), jnp.float32)
    batch3 = batch.astype(jnp.int32).reshape(NBLK, 1, BN)

    hc = _encode(x, enc_W, enc_b)
    for l in range(L):
        agg2 = _sc_agg(hc, src2, dst2, zeros)
        h2, ps, pq = _mlp(hc, agg2, eps[l], W1[l], b1[l], W2[l], b2[l])
        hc = _norm(h2, ps, pq, gamma[l], beta[l])
    return _pool_cls(hc, batch3, cls_W, cls_b)


# speedup vs baseline: 3.6035x; 1.7928x over previous
import functools

import jax
import jax.numpy as jnp
from jax import lax
from jax.experimental import pallas as pl
from jax.experimental.pallas import tpu as pltpu
from jax.experimental.pallas import tpu_sc as plsc

N = 100000
E = 1600000
F = 128
H = 32
G = 64
L = 4
C = 2
HH = H // 2
P = 8
N2 = 102400
NP = N2 // P
NPREAL = N // P
PADN = N2 - N

SROWS = N2 + 64
CH = 128
TILES = 16
NROW = 12544
EPAD = NROW * CH - E
RPT = NROW // TILES
IDXB = 56
NOB = RPT // IDXB
ZPT = SROWS // TILES
WPT = N2 // TILES


def _sc_agg_body(hc_hbm, src_hbm, dst_hbm, zeros_hbm, out_hbm,
                 srcb, dstd, rows, aggsh, gsem, ssem):
    c = lax.axis_index("c")
    s = lax.axis_index("s")

    z0 = s * ZPT
    pltpu.sync_copy(zeros_hbm.at[pl.ds(z0, ZPT)], aggsh.at[pl.ds(z0, ZPT)])
    plsc.subcore_barrier()

    table = hc_hbm.at[c]

    @pl.loop(0, NOB)
    def _ob(ob):
        r0 = s * RPT + ob * IDXB
        pltpu.sync_copy(src_hbm.at[pl.ds(r0, IDXB)], srcb)
        pltpu.sync_copy(dst_hbm.at[pl.ds(r0, IDXB)], dstd)

        for p in range(3):
            pltpu.async_copy(table.at[srcb.at[p]], rows.at[p], gsem.at[p])

        @pl.loop(0, IDXB)
        def _go(j):
            b = j & 3
            pltpu.make_async_copy(table.at[srcb.at[j]], rows.at[b],
                                  gsem.at[b]).wait()
            pltpu.async_copy(rows.at[b], aggsh.at[dstd.at[j]],
                             ssem.at[b], add=True)

            @pl.when(j < IDXB - 3)
            def _():
                bn = (j + 3) & 3

                @pl.when(j >= 1)
                def _():
                    pltpu.make_async_copy(rows.at[bn],
                                          aggsh.at[dstd.at[j - 1]],
                                          ssem.at[bn]).wait()

                pltpu.async_copy(table.at[srcb.at[j + 3]], rows.at[bn],
                                 gsem.at[bn])

        for q in range(4):
            jq = IDXB - 4 + q
            pltpu.make_async_copy(rows.at[jq & 3],
                                  aggsh.at[dstd.at[jq]],
                                  ssem.at[jq & 3]).wait()

    plsc.subcore_barrier()
    w0 = s * WPT
    pltpu.sync_copy(aggsh.at[pl.ds(w0, WPT)],
                    out_hbm.at[c].at[pl.ds(w0, WPT)])


_sc_agg_cached = None


def _sc_agg(hc, src2, dst2, zeros):
    global _sc_agg_cached
    if _sc_agg_cached is None:
        mesh = plsc.VectorSubcoreMesh(core_axis_name="c",
                                      subcore_axis_name="s",
                                      num_cores=2, num_subcores=TILES)
        _sc_agg_cached = functools.partial(
            pl.kernel,
            out_type=jax.ShapeDtypeStruct((2, N2, HH), jnp.float32),
            mesh=mesh,
            compiler_params=pltpu.CompilerParams(use_tc_tiling_on_sc=False),
            scratch_types=[
                pltpu.VMEM((IDXB, CH), jnp.int32),
                pltpu.VMEM((IDXB, CH), jnp.int32),
                pltpu.VMEM((4, CH, HH), jnp.float32),
                pltpu.VMEM_SHARED((SROWS, HH), jnp.float32),
                pltpu.SemaphoreType.DMA((4,)),
                pltpu.SemaphoreType.DMA((4,)),
            ],
        )(_sc_agg_body)
    return _sc_agg_cached(hc, src2, dst2, zeros)


PG = 128
PCH = 128
PNR = N2 // PCH
PRT = PNR // TILES


def _sc_pool_body(hc_hbm, bidx_hbm, zerosp_hbm, out_hbm,
                  bidxb, rows, aggsh, gsem, ssem):
    c = lax.axis_index("c")
    s = lax.axis_index("s")

    z0 = s * (PG // TILES)
    pltpu.sync_copy(zerosp_hbm.at[pl.ds(z0, PG // TILES)],
                    aggsh.at[pl.ds(z0, PG // TILES)])
    r0 = s * PRT
    pltpu.sync_copy(bidx_hbm.at[pl.ds(r0, PRT)], bidxb)
    plsc.subcore_barrier()

    table = hc_hbm.at[c]

    for p in range(3):
        pltpu.async_copy(table.at[pl.ds((r0 + p) * PCH, PCH)], rows.at[p],
                         gsem.at[p])

    @pl.loop(0, PRT)
    def _go(j):
        b = j & 3
        pltpu.make_async_copy(table.at[pl.ds((r0 + j) * PCH, PCH)],
                              rows.at[b], gsem.at[b]).wait()
        pltpu.async_copy(rows.at[b], aggsh.at[bidxb.at[j]],
                         ssem.at[b], add=True)

        @pl.when(j < PRT - 3)
        def _():
            bn = (j + 3) & 3

            @pl.when(j >= 1)
            def _():
                pltpu.make_async_copy(rows.at[bn],
                                      aggsh.at[bidxb.at[j - 1]],
                                      ssem.at[bn]).wait()

            pltpu.async_copy(table.at[pl.ds((r0 + j + 3) * PCH, PCH)],
                             rows.at[bn], gsem.at[bn])

    for q in range(4):
        jq = PRT - 4 + q
        pltpu.make_async_copy(rows.at[jq & 3], aggsh.at[bidxb.at[jq]],
                              ssem.at[jq & 3]).wait()

    plsc.subcore_barrier()

    @pl.when(s == 0)
    def _():
        pltpu.sync_copy(aggsh.at[pl.ds(0, G)], out_hbm.at[c])


_sc_pool_cached = None


def _sc_pool(hc, bidx, zerosp):
    global _sc_pool_cached
    if _sc_pool_cached is None:
        mesh = plsc.VectorSubcoreMesh(core_axis_name="c",
                                      subcore_axis_name="s",
                                      num_cores=2, num_subcores=TILES)
        _sc_pool_cached = functools.partial(
            pl.kernel,
            out_type=jax.ShapeDtypeStruct((2, G, HH), jnp.float32),
            mesh=mesh,
            compiler_params=pltpu.CompilerParams(use_tc_tiling_on_sc=False),
            scratch_types=[
                pltpu.VMEM((PRT, PCH), jnp.int32),
                pltpu.VMEM((4, PCH, HH), jnp.float32),
                pltpu.VMEM_SHARED((PG, HH), jnp.float32),
                pltpu.SemaphoreType.DMA((4,)),
                pltpu.SemaphoreType.DMA((4,)),
            ],
        )(_sc_pool_body)
    return _sc_pool_cached(hc, bidx, zerosp)


BP = 512
NBLK = NP // BP


def _enc_body(xr, wr, br, or_):
    o0 = jnp.dot(xr[...], wr[0], preferred_element_type=jnp.float32) + br[0]
    o1 = jnp.dot(xr[...], wr[1], preferred_element_type=jnp.float32) + br[1]
    or_[...] = jnp.stack([o0, o1])


def _encode(x8, wenc, benc):
    return pl.pallas_call(
        _enc_body,
        grid=(NBLK,),
        in_specs=[pl.BlockSpec((BP, P * F), lambda i: (i, 0)),
                  pl.BlockSpec((2, P * F, P * HH), lambda i: (0, 0, 0)),
                  pl.BlockSpec((2, 1, P * HH), lambda i: (0, 0, 0))],
        out_specs=pl.BlockSpec((2, BP, P * HH), lambda i: (0, i, 0)),
        out_shape=jax.ShapeDtypeStruct((2, NP, P * HH), jnp.float32),
    )(x8, wenc, benc)


def _mlp_body(eps_ref, hr, ar, w1r, b1r, w2r, b2r, h2r, psr, pqr):
    i = pl.program_id(0)
    e = 1.0 + eps_ref[0]
    z0 = hr[0] * e + ar[0]
    z1 = hr[1] * e + ar[1]

    def mm(a0, a1, w, b, da):
        return (jnp.dot(a0, w[0, da], preferred_element_type=jnp.float32)
                + jnp.dot(a1, w[1, da], preferred_element_type=jnp.float32)
                + b[da])

    u0 = jnp.maximum(mm(z0, z1, w1r, b1r, 0), 0.0)
    u1 = jnp.maximum(mm(z0, z1, w1r, b1r, 1), 0.0)
    grow = i * BP + lax.broadcasted_iota(jnp.int32, (BP, P * HH), 0)
    mask = (grow < NPREAL).astype(jnp.float32)
    h20 = mm(u0, u1, w2r, b2r, 0) * mask
    h21 = mm(u0, u1, w2r, b2r, 1) * mask
    h2r[...] = jnp.stack([h20, h21])

    @pl.when(i == 0)
    def _():
        psr[...] = jnp.zeros_like(psr)
        pqr[...] = jnp.zeros_like(pqr)

    psr[...] += jnp.stack([jnp.sum(h20, axis=0, keepdims=True),
                           jnp.sum(h21, axis=0, keepdims=True)])
    pqr[...] += jnp.stack([jnp.sum(h20 * h20, axis=0, keepdims=True),
                           jnp.sum(h21 * h21, axis=0, keepdims=True)])


def _mlp(hcp, aggp, eps_l, w1d, b1t, w2d, b2t):
    lanes = P * HH
    return pl.pallas_call(
        _mlp_body,
        grid=(NBLK,),
        in_specs=[pl.BlockSpec(memory_space=pltpu.SMEM),
                  pl.BlockSpec((2, BP, lanes), lambda i: (0, i, 0)),
                  pl.BlockSpec((2, BP, lanes), lambda i: (0, i, 0)),
                  pl.BlockSpec((2, 2, lanes, lanes), lambda i: (0, 0, 0, 0)),
                  pl.BlockSpec((2, 1, lanes), lambda i: (0, 0, 0)),
                  pl.BlockSpec((2, 2, lanes, lanes), lambda i: (0, 0, 0, 0)),
                  pl.BlockSpec((2, 1, lanes), lambda i: (0, 0, 0))],
        out_specs=[pl.BlockSpec((2, BP, lanes), lambda i: (0, i, 0)),
                   pl.BlockSpec((2, 1, lanes), lambda i: (0, 0, 0)),
                   pl.BlockSpec((2, 1, lanes), lambda i: (0, 0, 0))],
        out_shape=[jax.ShapeDtypeStruct((2, NP, lanes), jnp.float32),
                   jax.ShapeDtypeStruct((2, 1, lanes), jnp.float32),
                   jax.ShapeDtypeStruct((2, 1, lanes), jnp.float32)],
    )(eps_l.reshape(1), hcp, aggp, w1d, b1t, w2d, b2t)


def _norm_body(h2r, psr, pqr, kr, gr, br, or_):
    outs = []
    for a in range(2):
        s = jnp.dot(psr[a], kr[...], preferred_element_type=jnp.float32)
        q = jnp.dot(pqr[a], kr[...], preferred_element_type=jnp.float32)
        mean = s * (1.0 / N)
        var = q * (1.0 / N) - mean * mean
        scale = gr[a] * lax.rsqrt(var + 1e-5)
        outs.append(jnp.maximum((h2r[a] - mean) * scale + br[a], 0.0))
    or_[...] = jnp.stack(outs)


def _norm(h2p, ps, pq, kfold, gt, bt):
    lanes = P * HH
    return pl.pallas_call(
        _norm_body,
        grid=(NBLK,),
        in_specs=[pl.BlockSpec((2, BP, lanes), lambda i: (0, i, 0)),
                  pl.BlockSpec((2, 1, lanes), lambda i: (0, 0, 0)),
                  pl.BlockSpec((2, 1, lanes), lambda i: (0, 0, 0)),
                  pl.BlockSpec((lanes, lanes), lambda i: (0, 0)),
                  pl.BlockSpec((2, 1, lanes), lambda i: (0, 0, 0)),
                  pl.BlockSpec((2, 1, lanes), lambda i: (0, 0, 0))],
        out_specs=pl.BlockSpec((2, BP, lanes), lambda i: (0, i, 0)),
        out_shape=jax.ShapeDtypeStruct((2, NP, lanes), jnp.float32),
    )(h2p, ps, pq, kfold, gt, bt)


def _cls_body(pr, wr, br, or_):
    hcat = jnp.concatenate([pr[0], pr[1]], axis=1)
    or_[...] = jnp.dot(hcat, wr[...],
                       preferred_element_type=jnp.float32) + br[...]


def _cls(pooled, cls_W, cls_b):
    return pl.pallas_call(
        _cls_body,
        in_specs=[pl.BlockSpec((2, G, HH), lambda: (0, 0, 0)),
                  pl.BlockSpec((H, C), lambda: (0, 0)),
                  pl.BlockSpec((1, C), lambda: (0, 0))],
        out_specs=pl.BlockSpec((G, C), lambda: (0, 0)),
        out_shape=jax.ShapeDtypeStruct((G, C), jnp.float32),
    )(pooled, cls_W, cls_b.reshape(1, C))


def _kron8(m):
    return jnp.kron(jnp.eye(P, dtype=jnp.float32), m)


def kernel(x, edge_index, batch, enc_W, enc_b, eps, W1, b1, W2, b2,
           gamma, beta, cls_W, cls_b):
    src2 = jnp.concatenate(
        [edge_index[0], jnp.zeros((EPAD,), jnp.int32)]).reshape(NROW, CH)
    dst2 = jnp.concatenate(
        [edge_index[1], jnp.full((EPAD,), N2, jnp.int32)]).reshape(NROW, CH)
    zeros = jnp.zeros((SROWS, HH), jnp.float32)
    zerosp = jnp.zeros((PG, HH), jnp.float32)
    bidx = jnp.concatenate(
        [batch.astype(jnp.int32), jnp.full((PADN,), G, jnp.int32)]
    ).reshape(PNR, PCH)
    x8 = jnp.concatenate(
        [x, jnp.zeros((PADN, F), jnp.float32)]).reshape(NP, P * F)

    wenc = jnp.stack([_kron8(enc_W[:, :HH]), _kron8(enc_W[:, HH:])])
    benc = jnp.stack([jnp.tile(enc_b[:HH], P), jnp.tile(enc_b[HH:], P)])
    benc = benc.reshape(2, 1, P * HH)
    lanes = P * HH
    iota = jnp.arange(lanes)
    kfold = (iota[:, None] % HH == iota[None, :] % HH).astype(jnp.float32)

    def packmat(W_l):
        return jnp.stack([
            jnp.stack([_kron8(W_l[:HH, :HH]), _kron8(W_l[:HH, HH:])]),
            jnp.stack([_kron8(W_l[HH:, :HH]), _kron8(W_l[HH:, HH:])]),
        ])

    def packvec(v_l):
        return jnp.stack([jnp.tile(v_l[:HH], P),
                          jnp.tile(v_l[HH:], P)]).reshape(2, 1, lanes)

    hcp = _encode(x8, wenc, benc)
    for l in range(L):
        agg2 = _sc_agg(hcp.reshape(2, N2, HH), src2, dst2, zeros)
        h2p, ps, pq = _mlp(hcp, agg2.reshape(2, NP, lanes), eps[l],
                           packmat(W1[l]), packvec(b1[l]),
                           packmat(W2[l]), packvec(b2[l]))
        hcp = _norm(h2p, ps, pq, kfold, packvec(gamma[l]), packvec(beta[l]))
    pooled = _sc_pool(hcp.reshape(2, N2, HH), bidx, zerosp)
    return _cls(pooled, cls_W, cls_b)


# speedup vs baseline: 4.2768x; 1.1868x over previous
import functools

import jax
import jax.numpy as jnp
from jax import lax
from jax.experimental import pallas as pl
from jax.experimental.pallas import tpu as pltpu
from jax.experimental.pallas import tpu_sc as plsc

N = 100000
E = 1600000
F = 128
H = 32
G = 64
L = 4
C = 2
HH = H // 2
P = 8
N2 = 102400
NP = N2 // P
NPREAL = N // P
PADN = N2 - N

SROWS = N2 + 64
CH = 128
TILES = 16
NROW = 12544
EPAD = NROW * CH - E
RPT = NROW // TILES
IDXB = 56
NOB = RPT // IDXB
RB = 6
ZPT = SROWS // TILES
WPT = N2 // TILES


def _sc_agg_body(hc_hbm, src_hbm, dst_hbm, zeros_hbm, out_hbm,
                 srcb, dstd, rows, aggsh, gsem, ssem):
    c = lax.axis_index("c")
    s = lax.axis_index("s")

    z0 = s * ZPT
    pltpu.sync_copy(zeros_hbm.at[pl.ds(z0, ZPT)], aggsh.at[pl.ds(z0, ZPT)])
    plsc.subcore_barrier()

    table = hc_hbm.at[c]

    @pl.loop(0, NOB)
    def _ob(ob):
        r0 = s * RPT + ob * IDXB
        pltpu.sync_copy(src_hbm.at[pl.ds(r0, IDXB)], srcb)
        pltpu.sync_copy(dst_hbm.at[pl.ds(r0, IDXB)], dstd)

        for p in range(RB - 1):
            pltpu.async_copy(table.at[srcb.at[p]], rows.at[p], gsem.at[p])

        @pl.loop(0, IDXB)
        def _go(j):
            b = lax.rem(j, RB)
            pltpu.make_async_copy(table.at[srcb.at[j]], rows.at[b],
                                  gsem.at[b]).wait()
            pltpu.async_copy(rows.at[b], aggsh.at[dstd.at[j]],
                             ssem.at[b], add=True)

            @pl.when(j < IDXB - (RB - 1))
            def _():
                bn = lax.rem(j + RB - 1, RB)

                @pl.when(j >= 1)
                def _():
                    pltpu.make_async_copy(rows.at[bn],
                                          aggsh.at[dstd.at[j - 1]],
                                          ssem.at[bn]).wait()

                pltpu.async_copy(table.at[srcb.at[j + RB - 1]], rows.at[bn],
                                 gsem.at[bn])

        for q in range(RB):
            jq = IDXB - RB + q
            pltpu.make_async_copy(rows.at[jq % RB],
                                  aggsh.at[dstd.at[jq]],
                                  ssem.at[jq % RB]).wait()

    plsc.subcore_barrier()
    w0 = s * WPT
    pltpu.sync_copy(aggsh.at[pl.ds(w0, WPT)],
                    out_hbm.at[c].at[pl.ds(w0, WPT)])


_sc_agg_cached = None


def _sc_agg(hc, src2, dst2, zeros):
    global _sc_agg_cached
    if _sc_agg_cached is None:
        mesh = plsc.VectorSubcoreMesh(core_axis_name="c",
                                      subcore_axis_name="s",
                                      num_cores=2, num_subcores=TILES)
        _sc_agg_cached = functools.partial(
            pl.kernel,
            out_type=jax.ShapeDtypeStruct((2, N2, HH), jnp.float32),
            mesh=mesh,
            compiler_params=pltpu.CompilerParams(use_tc_tiling_on_sc=False),
            scratch_types=[
                pltpu.VMEM((IDXB, CH), jnp.int32),
                pltpu.VMEM((IDXB, CH), jnp.int32),
                pltpu.VMEM((RB, CH, HH), jnp.float32),
                pltpu.VMEM_SHARED((SROWS, HH), jnp.float32),
                pltpu.SemaphoreType.DMA((RB,)),
                pltpu.SemaphoreType.DMA((RB,)),
            ],
        )(_sc_agg_body)
    return _sc_agg_cached(hc, src2, dst2, zeros)


PG = 128
PCH = 128
PNR = N2 // PCH
PRT = PNR // TILES


def _sc_pool_body(hc_hbm, bidx_hbm, zerosp_hbm, out_hbm,
                  bidxb, rows, aggsh, gsem, ssem):
    c = lax.axis_index("c")
    s = lax.axis_index("s")

    z0 = s * (PG // TILES)
    pltpu.sync_copy(zerosp_hbm.at[pl.ds(z0, PG // TILES)],
                    aggsh.at[pl.ds(z0, PG // TILES)])
    r0 = s * PRT
    pltpu.sync_copy(bidx_hbm.at[pl.ds(r0, PRT)], bidxb)
    plsc.subcore_barrier()

    table = hc_hbm.at[c]

    for p in range(3):
        pltpu.async_copy(table.at[pl.ds((r0 + p) * PCH, PCH)], rows.at[p],
                         gsem.at[p])

    @pl.loop(0, PRT)
    def _go(j):
        b = j & 3
        pltpu.make_async_copy(table.at[pl.ds((r0 + j) * PCH, PCH)],
                              rows.at[b], gsem.at[b]).wait()
        pltpu.async_copy(rows.at[b], aggsh.at[bidxb.at[j]],
                         ssem.at[b], add=True)

        @pl.when(j < PRT - 3)
        def _():
            bn = (j + 3) & 3

            @pl.when(j >= 1)
            def _():
                pltpu.make_async_copy(rows.at[bn],
                                      aggsh.at[bidxb.at[j - 1]],
                                      ssem.at[bn]).wait()

            pltpu.async_copy(table.at[pl.ds((r0 + j + 3) * PCH, PCH)],
                             rows.at[bn], gsem.at[bn])

    for q in range(4):
        jq = PRT - 4 + q
        pltpu.make_async_copy(rows.at[jq & 3], aggsh.at[bidxb.at[jq]],
                              ssem.at[jq & 3]).wait()

    plsc.subcore_barrier()

    @pl.when(s == 0)
    def _():
        pltpu.sync_copy(aggsh.at[pl.ds(0, G)], out_hbm.at[c])


_sc_pool_cached = None


def _sc_pool(hc, bidx, zerosp):
    global _sc_pool_cached
    if _sc_pool_cached is None:
        mesh = plsc.VectorSubcoreMesh(core_axis_name="c",
                                      subcore_axis_name="s",
                                      num_cores=2, num_subcores=TILES)
        _sc_pool_cached = functools.partial(
            pl.kernel,
            out_type=jax.ShapeDtypeStruct((2, G, HH), jnp.float32),
            mesh=mesh,
            compiler_params=pltpu.CompilerParams(use_tc_tiling_on_sc=False),
            scratch_types=[
                pltpu.VMEM((PRT, PCH), jnp.int32),
                pltpu.VMEM((4, PCH, HH), jnp.float32),
                pltpu.VMEM_SHARED((PG, HH), jnp.float32),
                pltpu.SemaphoreType.DMA((4,)),
                pltpu.SemaphoreType.DMA((4,)),
            ],
        )(_sc_pool_body)
    return _sc_pool_cached(hc, bidx, zerosp)


BP = 512
NBLK = NP // BP


def _enc_body(xr, wr, br, or_):
    o0 = jnp.dot(xr[...], wr[0], preferred_element_type=jnp.float32) + br[0]
    o1 = jnp.dot(xr[...], wr[1], preferred_element_type=jnp.float32) + br[1]
    or_[...] = jnp.stack([o0, o1])


def _encode(x8, wenc, benc):
    return pl.pallas_call(
        _enc_body,
        grid=(NBLK,),
        in_specs=[pl.BlockSpec((BP, P * F), lambda i: (i, 0)),
                  pl.BlockSpec((2, P * F, P * HH), lambda i: (0, 0, 0)),
                  pl.BlockSpec((2, 1, P * HH), lambda i: (0, 0, 0))],
        out_specs=pl.BlockSpec((2, BP, P * HH), lambda i: (0, i, 0)),
        out_shape=jax.ShapeDtypeStruct((2, NP, P * HH), jnp.float32),
    )(x8, wenc, benc)


def _mlp_body(eps_ref, hr, ar, w1r, b1r, w2r, b2r, h2r, psr, pqr):
    i = pl.program_id(0)
    e = 1.0 + eps_ref[0]
    z0 = hr[0] * e + ar[0]
    z1 = hr[1] * e + ar[1]

    def mm(a0, a1, w, b, da):
        return (jnp.dot(a0, w[0, da], preferred_element_type=jnp.float32)
                + jnp.dot(a1, w[1, da], preferred_element_type=jnp.float32)
                + b[da])

    u0 = jnp.maximum(mm(z0, z1, w1r, b1r, 0), 0.0)
    u1 = jnp.maximum(mm(z0, z1, w1r, b1r, 1), 0.0)
    grow = i * BP + lax.broadcasted_iota(jnp.int32, (BP, P * HH), 0)
    mask = (grow < NPREAL).astype(jnp.float32)
    h20 = mm(u0, u1, w2r, b2r, 0) * mask
    h21 = mm(u0, u1, w2r, b2r, 1) * mask
    h2r[...] = jnp.stack([h20, h21])

    @pl.when(i == 0)
    def _():
        psr[...] = jnp.zeros_like(psr)
        pqr[...] = jnp.zeros_like(pqr)

    psr[...] += jnp.stack([jnp.sum(h20, axis=0, keepdims=True),
                           jnp.sum(h21, axis=0, keepdims=True)])
    pqr[...] += jnp.stack([jnp.sum(h20 * h20, axis=0, keepdims=True),
                           jnp.sum(h21 * h21, axis=0, keepdims=True)])


def _mlp(hcp, aggp, eps_l, w1d, b1t, w2d, b2t):
    lanes = P * HH
    return pl.pallas_call(
        _mlp_body,
        grid=(NBLK,),
        in_specs=[pl.BlockSpec(memory_space=pltpu.SMEM),
                  pl.BlockSpec((2, BP, lanes), lambda i: (0, i, 0)),
                  pl.BlockSpec((2, BP, lanes), lambda i: (0, i, 0)),
                  pl.BlockSpec((2, 2, lanes, lanes), lambda i: (0, 0, 0, 0)),
                  pl.BlockSpec((2, 1, lanes), lambda i: (0, 0, 0)),
                  pl.BlockSpec((2, 2, lanes, lanes), lambda i: (0, 0, 0, 0)),
                  pl.BlockSpec((2, 1, lanes), lambda i: (0, 0, 0))],
        out_specs=[pl.BlockSpec((2, BP, lanes), lambda i: (0, i, 0)),
                   pl.BlockSpec((2, 1, lanes), lambda i: (0, 0, 0)),
                   pl.BlockSpec((2, 1, lanes), lambda i: (0, 0, 0))],
        out_shape=[jax.ShapeDtypeStruct((2, NP, lanes), jnp.float32),
                   jax.ShapeDtypeStruct((2, 1, lanes), jnp.float32),
                   jax.ShapeDtypeStruct((2, 1, lanes), jnp.float32)],
    )(eps_l.reshape(1), hcp, aggp, w1d, b1t, w2d, b2t)


def _norm_body(h2r, psr, pqr, kr, gr, br, or_):
    outs = []
    for a in range(2):
        s = jnp.dot(psr[a], kr[...], preferred_element_type=jnp.float32)
        q = jnp.dot(pqr[a], kr[...], preferred_element_type=jnp.float32)
        mean = s * (1.0 / N)
        var = q * (1.0 / N) - mean * mean
        scale = gr[a] * lax.rsqrt(var + 1e-5)
        outs.append(jnp.maximum((h2r[a] - mean) * scale + br[a], 0.0))
    or_[...] = jnp.stack(outs)


def _norm(h2p, ps, pq, kfold, gt, bt):
    lanes = P * HH
    return pl.pallas_call(
        _norm_body,
        grid=(NBLK,),
        in_specs=[pl.BlockSpec((2, BP, lanes), lambda i: (0, i, 0)),
                  pl.BlockSpec((2, 1, lanes), lambda i: (0, 0, 0)),
                  pl.BlockSpec((2, 1, lanes), lambda i: (0, 0, 0)),
                  pl.BlockSpec((lanes, lanes), lambda i: (0, 0)),
                  pl.BlockSpec((2, 1, lanes), lambda i: (0, 0, 0)),
                  pl.BlockSpec((2, 1, lanes), lambda i: (0, 0, 0))],
        out_specs=pl.BlockSpec((2, BP, lanes), lambda i: (0, i, 0)),
        out_shape=jax.ShapeDtypeStruct((2, NP, lanes), jnp.float32),
    )(h2p, ps, pq, kfold, gt, bt)


def _cls_body(pr, wr, br, or_):
    hcat = jnp.concatenate([pr[0], pr[1]], axis=1)
    or_[...] = jnp.dot(hcat, wr[...],
                       preferred_element_type=jnp.float32) + br[...]


def _cls(pooled, cls_W, cls_b):
    return pl.pallas_call(
        _cls_body,
        in_specs=[pl.BlockSpec((2, G, HH), lambda: (0, 0, 0)),
                  pl.BlockSpec((H, C), lambda: (0, 0)),
                  pl.BlockSpec((1, C), lambda: (0, 0))],
        out_specs=pl.BlockSpec((G, C), lambda: (0, 0)),
        out_shape=jax.ShapeDtypeStruct((G, C), jnp.float32),
    )(pooled, cls_W, cls_b.reshape(1, C))


def _kron8(m):
    return jnp.kron(jnp.eye(P, dtype=jnp.float32), m)


def kernel(x, edge_index, batch, enc_W, enc_b, eps, W1, b1, W2, b2,
           gamma, beta, cls_W, cls_b):
    src2 = jnp.concatenate(
        [edge_index[0], jnp.zeros((EPAD,), jnp.int32)]).reshape(NROW, CH)
    dst2 = jnp.concatenate(
        [edge_index[1], jnp.full((EPAD,), N2, jnp.int32)]).reshape(NROW, CH)
    zeros = jnp.zeros((SROWS, HH), jnp.float32)
    zerosp = jnp.zeros((PG, HH), jnp.float32)
    bidx = jnp.concatenate(
        [batch.astype(jnp.int32), jnp.full((PADN,), G, jnp.int32)]
    ).reshape(PNR, PCH)
    x8 = jnp.concatenate(
        [x, jnp.zeros((PADN, F), jnp.float32)]).reshape(NP, P * F)

    wenc = jnp.stack([_kron8(enc_W[:, :HH]), _kron8(enc_W[:, HH:])])
    benc = jnp.stack([jnp.tile(enc_b[:HH], P), jnp.tile(enc_b[HH:], P)])
    benc = benc.reshape(2, 1, P * HH)
    lanes = P * HH
    iota = jnp.arange(lanes)
    kfold = (iota[:, None] % HH == iota[None, :] % HH).astype(jnp.float32)

    def packmat(W_l):
        return jnp.stack([
            jnp.stack([_kron8(W_l[:HH, :HH]), _kron8(W_l[:HH, HH:])]),
            jnp.stack([_kron8(W_l[HH:, :HH]), _kron8(W_l[HH:, HH:])]),
        ])

    def packvec(v_l):
        return jnp.stack([jnp.tile(v_l[:HH], P),
                          jnp.tile(v_l[HH:], P)]).reshape(2, 1, lanes)

    hcp = _encode(x8, wenc, benc)
    for l in range(L):
        agg2 = _sc_agg(hcp.reshape(2, N2, HH), src2, dst2, zeros)
        h2p, ps, pq = _mlp(hcp, agg2.reshape(2, NP, lanes), eps[l],
                           packmat(W1[l]), packvec(b1[l]),
                           packmat(W2[l]), packvec(b2[l]))
        hcp = _norm(h2p, ps, pq, kfold, packvec(gamma[l]), packvec(beta[l]))
    pooled = _sc_pool(hcp.reshape(2, N2, HH), bidx, zerosp)
    return _cls(pooled, cls_W, cls_b)


# speedup vs baseline: 4.2832x; 1.0015x over previous
import functools

import jax
import jax.numpy as jnp
from jax import lax
from jax.experimental import pallas as pl
from jax.experimental.pallas import tpu as pltpu
from jax.experimental.pallas import tpu_sc as plsc

N = 100000
E = 1600000
F = 128
H = 32
G = 64
L = 4
C = 2
HH = H // 2
P = 8
N2 = 102400
NP = N2 // P
NPREAL = N // P
PADN = N2 - N

SROWS = N2 + 64
CH = 128
TILES = 16
NROW = 12544
EPAD = NROW * CH - E
RPT = NROW // TILES
IDXB = 28
NOB = RPT // IDXB
RB = 8
ZPT = SROWS // TILES
WPT = N2 // TILES


def _sc_agg_body(hc_hbm, src_hbm, dst_hbm, zeros_hbm, out_hbm,
                 srcb, dstd, rows, aggsh, gsem, ssem):
    c = lax.axis_index("c")
    s = lax.axis_index("s")

    z0 = s * ZPT
    pltpu.sync_copy(zeros_hbm.at[pl.ds(z0, ZPT)], aggsh.at[pl.ds(z0, ZPT)])
    plsc.subcore_barrier()

    table = hc_hbm.at[c]

    @pl.loop(0, NOB)
    def _ob(ob):
        r0 = s * RPT + ob * IDXB
        pltpu.sync_copy(src_hbm.at[pl.ds(r0, IDXB)], srcb)
        pltpu.sync_copy(dst_hbm.at[pl.ds(r0, IDXB)], dstd)

        for p in range(RB - 1):
            pltpu.async_copy(table.at[srcb.at[p]], rows.at[p], gsem.at[p])

        @pl.loop(0, IDXB)
        def _go(j):
            b = lax.rem(j, RB)
            pltpu.make_async_copy(table.at[srcb.at[j]], rows.at[b],
                                  gsem.at[b]).wait()
            pltpu.async_copy(rows.at[b], aggsh.at[dstd.at[j]],
                             ssem.at[b], add=True)

            @pl.when(j < IDXB - (RB - 1))
            def _():
                bn = lax.rem(j + RB - 1, RB)

                @pl.when(j >= 1)
                def _():
                    pltpu.make_async_copy(rows.at[bn],
                                          aggsh.at[dstd.at[j - 1]],
                                          ssem.at[bn]).wait()

                pltpu.async_copy(table.at[srcb.at[j + RB - 1]], rows.at[bn],
                                 gsem.at[bn])

        for q in range(RB):
            jq = IDXB - RB + q
            pltpu.make_async_copy(rows.at[jq % RB],
                                  aggsh.at[dstd.at[jq]],
                                  ssem.at[jq % RB]).wait()

    plsc.subcore_barrier()
    w0 = s * WPT
    pltpu.sync_copy(aggsh.at[pl.ds(w0, WPT)],
                    out_hbm.at[c].at[pl.ds(w0, WPT)])


_sc_agg_cached = None


def _sc_agg(hc, src2, dst2, zeros):
    global _sc_agg_cached
    if _sc_agg_cached is None:
        mesh = plsc.VectorSubcoreMesh(core_axis_name="c",
                                      subcore_axis_name="s",
                                      num_cores=2, num_subcores=TILES)
        _sc_agg_cached = functools.partial(
            pl.kernel,
            out_type=jax.ShapeDtypeStruct((2, N2, HH), jnp.float32),
            mesh=mesh,
            compiler_params=pltpu.CompilerParams(use_tc_tiling_on_sc=False),
            scratch_types=[
                pltpu.VMEM((IDXB, CH), jnp.int32),
                pltpu.VMEM((IDXB, CH), jnp.int32),
                pltpu.VMEM((RB, CH, HH), jnp.float32),
                pltpu.VMEM_SHARED((SROWS, HH), jnp.float32),
                pltpu.SemaphoreType.DMA((RB,)),
                pltpu.SemaphoreType.DMA((RB,)),
            ],
        )(_sc_agg_body)
    return _sc_agg_cached(hc, src2, dst2, zeros)


PG = 128
PCH = 128
PNR = N2 // PCH
PRT = PNR // TILES


def _sc_pool_body(hc_hbm, bidx_hbm, zerosp_hbm, out_hbm,
                  bidxb, rows, aggsh, gsem, ssem):
    c = lax.axis_index("c")
    s = lax.axis_index("s")

    z0 = s * (PG // TILES)
    pltpu.sync_copy(zerosp_hbm.at[pl.ds(z0, PG // TILES)],
                    aggsh.at[pl.ds(z0, PG // TILES)])
    r0 = s * PRT
    pltpu.sync_copy(bidx_hbm.at[pl.ds(r0, PRT)], bidxb)
    plsc.subcore_barrier()

    table = hc_hbm.at[c]

    for p in range(3):
        pltpu.async_copy(table.at[pl.ds((r0 + p) * PCH, PCH)], rows.at[p],
                         gsem.at[p])

    @pl.loop(0, PRT)
    def _go(j):
        b = j & 3
        pltpu.make_async_copy(table.at[pl.ds((r0 + j) * PCH, PCH)],
                              rows.at[b], gsem.at[b]).wait()
        pltpu.async_copy(rows.at[b], aggsh.at[bidxb.at[j]],
                         ssem.at[b], add=True)

        @pl.when(j < PRT - 3)
        def _():
            bn = (j + 3) & 3

            @pl.when(j >= 1)
            def _():
                pltpu.make_async_copy(rows.at[bn],
                                      aggsh.at[bidxb.at[j - 1]],
                                      ssem.at[bn]).wait()

            pltpu.async_copy(table.at[pl.ds((r0 + j + 3) * PCH, PCH)],
                             rows.at[bn], gsem.at[bn])

    for q in range(4):
        jq = PRT - 4 + q
        pltpu.make_async_copy(rows.at[jq & 3], aggsh.at[bidxb.at[jq]],
                              ssem.at[jq & 3]).wait()

    plsc.subcore_barrier()

    @pl.when(s == 0)
    def _():
        pltpu.sync_copy(aggsh.at[pl.ds(0, G)], out_hbm.at[c])


_sc_pool_cached = None


def _sc_pool(hc, bidx, zerosp):
    global _sc_pool_cached
    if _sc_pool_cached is None:
        mesh = plsc.VectorSubcoreMesh(core_axis_name="c",
                                      subcore_axis_name="s",
                                      num_cores=2, num_subcores=TILES)
        _sc_pool_cached = functools.partial(
            pl.kernel,
            out_type=jax.ShapeDtypeStruct((2, G, HH), jnp.float32),
            mesh=mesh,
            compiler_params=pltpu.CompilerParams(use_tc_tiling_on_sc=False),
            scratch_types=[
                pltpu.VMEM((PRT, PCH), jnp.int32),
                pltpu.VMEM((4, PCH, HH), jnp.float32),
                pltpu.VMEM_SHARED((PG, HH), jnp.float32),
                pltpu.SemaphoreType.DMA((4,)),
                pltpu.SemaphoreType.DMA((4,)),
            ],
        )(_sc_pool_body)
    return _sc_pool_cached(hc, bidx, zerosp)


BP = 512
NBLK = NP // BP


def _enc_body(xr, wr, br, or_):
    o0 = jnp.dot(xr[...], wr[0], preferred_element_type=jnp.float32) + br[0]
    o1 = jnp.dot(xr[...], wr[1], preferred_element_type=jnp.float32) + br[1]
    or_[...] = jnp.stack([o0, o1])


def _encode(x8, wenc, benc):
    return pl.pallas_call(
        _enc_body,
        grid=(NBLK,),
        in_specs=[pl.BlockSpec((BP, P * F), lambda i: (i, 0)),
                  pl.BlockSpec((2, P * F, P * HH), lambda i: (0, 0, 0)),
                  pl.BlockSpec((2, 1, P * HH), lambda i: (0, 0, 0))],
        out_specs=pl.BlockSpec((2, BP, P * HH), lambda i: (0, i, 0)),
        out_shape=jax.ShapeDtypeStruct((2, NP, P * HH), jnp.float32),
    )(x8, wenc, benc)


def _mlp_body(eps_ref, hr, ar, w1r, b1r, w2r, b2r, h2r, psr, pqr):
    i = pl.program_id(0)
    e = 1.0 + eps_ref[0]
    z0 = hr[0] * e + ar[0]
    z1 = hr[1] * e + ar[1]

    def mm(a0, a1, w, b, da):
        return (jnp.dot(a0, w[0, da], preferred_element_type=jnp.float32)
                + jnp.dot(a1, w[1, da], preferred_element_type=jnp.float32)
                + b[da])

    u0 = jnp.maximum(mm(z0, z1, w1r, b1r, 0), 0.0)
    u1 = jnp.maximum(mm(z0, z1, w1r, b1r, 1), 0.0)
    grow = i * BP + lax.broadcasted_iota(jnp.int32, (BP, P * HH), 0)
    mask = (grow < NPREAL).astype(jnp.float32)
    h20 = mm(u0, u1, w2r, b2r, 0) * mask
    h21 = mm(u0, u1, w2r, b2r, 1) * mask
    h2r[...] = jnp.stack([h20, h21])

    @pl.when(i == 0)
    def _():
        psr[...] = jnp.zeros_like(psr)
        pqr[...] = jnp.zeros_like(pqr)

    psr[...] += jnp.stack([jnp.sum(h20, axis=0, keepdims=True),
                           jnp.sum(h21, axis=0, keepdims=True)])
    pqr[...] += jnp.stack([jnp.sum(h20 * h20, axis=0, keepdims=True),
                           jnp.sum(h21 * h21, axis=0, keepdims=True)])


def _mlp(hcp, aggp, eps_l, w1d, b1t, w2d, b2t):
    lanes = P * HH
    return pl.pallas_call(
        _mlp_body,
        grid=(NBLK,),
        in_specs=[pl.BlockSpec(memory_space=pltpu.SMEM),
                  pl.BlockSpec((2, BP, lanes), lambda i: (0, i, 0)),
                  pl.BlockSpec((2, BP, lanes), lambda i: (0, i, 0)),
                  pl.BlockSpec((2, 2, lanes, lanes), lambda i: (0, 0, 0, 0)),
                  pl.BlockSpec((2, 1, lanes), lambda i: (0, 0, 0)),
                  pl.BlockSpec((2, 2, lanes, lanes), lambda i: (0, 0, 0, 0)),
                  pl.BlockSpec((2, 1, lanes), lambda i: (0, 0, 0))],
        out_specs=[pl.BlockSpec((2, BP, lanes), lambda i: (0, i, 0)),
                   pl.BlockSpec((2, 1, lanes), lambda i: (0, 0, 0)),
                   pl.BlockSpec((2, 1, lanes), lambda i: (0, 0, 0))],
        out_shape=[jax.ShapeDtypeStruct((2, NP, lanes), jnp.float32),
                   jax.ShapeDtypeStruct((2, 1, lanes), jnp.float32),
                   jax.ShapeDtypeStruct((2, 1, lanes), jnp.float32)],
    )(eps_l.reshape(1), hcp, aggp, w1d, b1t, w2d, b2t)


def _norm_body(h2r, psr, pqr, kr, gr, br, or_):
    outs = []
    for a in range(2):
        s = jnp.dot(psr[a], kr[...], preferred_element_type=jnp.float32)
        q = jnp.dot(pqr[a], kr[...], preferred_element_type=jnp.float32)
        mean = s * (1.0 / N)
        var = q * (1.0 / N) - mean * mean
        scale = gr[a] * lax.rsqrt(var + 1e-5)
        outs.append(jnp.maximum((h2r[a] - mean) * scale + br[a], 0.0))
    or_[...] = jnp.stack(outs)


def _norm(h2p, ps, pq, kfold, gt, bt):
    lanes = P * HH
    return pl.pallas_call(
        _norm_body,
        grid=(NBLK,),
        in_specs=[pl.BlockSpec((2, BP, lanes), lambda i: (0, i, 0)),
                  pl.BlockSpec((2, 1, lanes), lambda i: (0, 0, 0)),
                  pl.BlockSpec((2, 1, lanes), lambda i: (0, 0, 0)),
                  pl.BlockSpec((lanes, lanes), lambda i: (0, 0)),
                  pl.BlockSpec((2, 1, lanes), lambda i: (0, 0, 0)),
                  pl.BlockSpec((2, 1, lanes), lambda i: (0, 0, 0))],
        out_specs=pl.BlockSpec((2, BP, lanes), lambda i: (0, i, 0)),
        out_shape=jax.ShapeDtypeStruct((2, NP, lanes), jnp.float32),
    )(h2p, ps, pq, kfold, gt, bt)


def _cls_body(pr, wr, br, or_):
    hcat = jnp.concatenate([pr[0], pr[1]], axis=1)
    or_[...] = jnp.dot(hcat, wr[...],
                       preferred_element_type=jnp.float32) + br[...]


def _cls(pooled, cls_W, cls_b):
    return pl.pallas_call(
        _cls_body,
        in_specs=[pl.BlockSpec((2, G, HH), lambda: (0, 0, 0)),
                  pl.BlockSpec((H, C), lambda: (0, 0)),
                  pl.BlockSpec((1, C), lambda: (0, 0))],
        out_specs=pl.BlockSpec((G, C), lambda: (0, 0)),
        out_shape=jax.ShapeDtypeStruct((G, C), jnp.float32),
    )(pooled, cls_W, cls_b.reshape(1, C))


def _kron8(m):
    return jnp.kron(jnp.eye(P, dtype=jnp.float32), m)


def kernel(x, edge_index, batch, enc_W, enc_b, eps, W1, b1, W2, b2,
           gamma, beta, cls_W, cls_b):
    src2 = jnp.concatenate(
        [edge_index[0], jnp.zeros((EPAD,), jnp.int32)]).reshape(NROW, CH)
    dst2 = jnp.concatenate(
        [edge_index[1], jnp.full((EPAD,), N2, jnp.int32)]).reshape(NROW, CH)
    zeros = jnp.zeros((SROWS, HH), jnp.float32)
    zerosp = jnp.zeros((PG, HH), jnp.float32)
    bidx = jnp.concatenate(
        [batch.astype(jnp.int32), jnp.full((PADN,), G, jnp.int32)]
    ).reshape(PNR, PCH)
    x8 = jnp.concatenate(
        [x, jnp.zeros((PADN, F), jnp.float32)]).reshape(NP, P * F)

    wenc = jnp.stack([_kron8(enc_W[:, :HH]), _kron8(enc_W[:, HH:])])
    benc = jnp.stack([jnp.tile(enc_b[:HH], P), jnp.tile(enc_b[HH:], P)])
    benc = benc.reshape(2, 1, P * HH)
    lanes = P * HH
    iota = jnp.arange(lanes)
    kfold = (iota[:, None] % HH == iota[None, :] % HH).astype(jnp.float32)

    def packmat(W_l):
        return jnp.stack([
            jnp.stack([_kron8(W_l[:HH, :HH]), _kron8(W_l[:HH, HH:])]),
            jnp.stack([_kron8(W_l[HH:, :HH]), _kron8(W_l[HH:, HH:])]),
        ])

    def packvec(v_l):
        return jnp.stack([jnp.tile(v_l[:HH], P),
                          jnp.tile(v_l[HH:], P)]).reshape(2, 1, lanes)

    hcp = _encode(x8, wenc, benc)
    for l in range(L):
        agg2 = _sc_agg(hcp.reshape(2, N2, HH), src2, dst2, zeros)
        h2p, ps, pq = _mlp(hcp, agg2.reshape(2, NP, lanes), eps[l],
                           packmat(W1[l]), packvec(b1[l]),
                           packmat(W2[l]), packvec(b2[l]))
        hcp = _norm(h2p, ps, pq, kfold, packvec(gamma[l]), packvec(beta[l]))
    pooled = _sc_pool(hcp.reshape(2, N2, HH), bidx, zerosp)
    return _cls(pooled, cls_W, cls_b)
